# baseline (speedup 1.0000x reference)
import functools

import jax
import jax.numpy as jnp
from jax import lax
from jax.experimental import pallas as pl
from jax.experimental.pallas import tpu as pltpu
from jax.experimental.pallas import tpu_sc as plsc

NC = 2
NS = 16
NW = NC * NS
CHUNK = 128
ZR = 128


RING_LCM = 40


def _sc_message_pass(table, srcp, dstp, zeros_blk, n_chunks):
    NP, F = table.shape
    rows_per_tile = NP // NS
    RING, LOOK = (10, 5) if F <= 32 else (8, 4)
    n_groups = n_chunks // RING
    mesh = plsc.VectorSubcoreMesh(core_axis_name="c", subcore_axis_name="s")

    @functools.partial(
        pl.kernel,
        out_type=jax.ShapeDtypeStruct((NC, NP, F), jnp.float32),
        mesh=mesh,
        scratch_types=[
            pltpu.VMEM_SHARED((NP, F), jnp.float32),
            pltpu.VMEM((n_chunks, CHUNK), jnp.int32),
            pltpu.VMEM((n_chunks, CHUNK), jnp.int32),
        ] + [pltpu.VMEM((CHUNK, F), jnp.float32) for _ in range(RING)]
          + [pltpu.SemaphoreType.DMA for _ in range(2 * RING)],
        compiler_params=pltpu.CompilerParams(use_tc_tiling_on_sc=False),
    )
    def body(table_hbm, src_hbm, dst_hbm, zeros_hbm, out_hbm,
             acc, src_v, dst_v, *bufs_and_sems):
        bufs = bufs_and_sems[:RING]
        gsem = bufs_and_sems[RING:2 * RING]
        ssem = bufs_and_sems[2 * RING:]
        c = lax.axis_index("c")
        s = lax.axis_index("s")
        wid = s * NC + c
        pltpu.sync_copy(src_hbm.at[wid], src_v)
        pltpu.sync_copy(dst_hbm.at[wid], dst_v)
        pltpu.sync_copy(zeros_hbm, bufs[0])
        base = s * rows_per_tile
        for r in range(rows_per_tile // ZR):
            pltpu.sync_copy(bufs[0], acc.at[pl.ds(base + r * ZR, ZR)])
        plsc.subcore_barrier()

        def issue_gather(k, b):
            pltpu.async_copy(table_hbm.at[src_v.at[k]], bufs[b], gsem[b])

        def wait_gather(k, b):
            pltpu.make_async_copy(table_hbm.at[src_v.at[k]], bufs[b],
                                  gsem[b]).wait()

        def issue_scatter(k, b):
            pltpu.async_copy(bufs[b], acc.at[dst_v.at[k]], ssem[b], add=True)

        def wait_scatter(k, b):
            pltpu.make_async_copy(bufs[b], acc.at[dst_v.at[k]],
                                  ssem[b]).wait()

        def group(g, first, last):
            base_k = g * RING
            for b in range(RING):
                k = base_k + b
                wait_gather(k, b)
                issue_scatter(k, b)
                bb = (b + LOOK) % RING
                if (not last) or (b < RING - LOOK):
                    if (not first) or (b >= RING - LOOK):
                        wait_scatter(k + LOOK - RING, bb)
                    issue_gather(k + LOOK, bb)

        for j in range(LOOK):
            issue_gather(j, j)
        group(0, True, n_groups == 1)
        if n_groups > 2:
            @pl.loop(1, n_groups - 1)
            def _(g):
                group(g, False, False)
        if n_groups > 1:
            group(n_groups - 1, False, True)
        for j in range(RING):
            wait_scatter(n_chunks - RING + j, j)

        plsc.subcore_barrier()
        pltpu.sync_copy(acc.at[pl.ds(base, rows_per_tile)],
                        out_hbm.at[c, pl.ds(base, rows_per_tile)])

    return body(table, srcp, dstp, zeros_blk)


def _sc_degree_pass(dstp, ones_blk, zeros_blk, NP, n_chunks):
    rows_per_tile = NP // NS
    mesh = plsc.VectorSubcoreMesh(core_axis_name="c", subcore_axis_name="s")

    NSEM = 4
    n_groups = n_chunks // NSEM

    @functools.partial(
        pl.kernel,
        out_type=jax.ShapeDtypeStruct((NC, NP, 16), jnp.float32),
        mesh=mesh,
        scratch_types=[
            pltpu.VMEM_SHARED((NP, 16), jnp.float32),
            pltpu.VMEM((n_chunks, CHUNK), jnp.int32),
            pltpu.VMEM((CHUNK, 16), jnp.float32),
            pltpu.VMEM((ZR, 16), jnp.float32),
        ] + [pltpu.SemaphoreType.DMA for _ in range(NSEM)],
        compiler_params=pltpu.CompilerParams(use_tc_tiling_on_sc=False),
    )
    def body(dst_hbm, ones_hbm, zeros_hbm, out_hbm, acc, dst_v, ones_v, zer_v,
             *sems):
        c = lax.axis_index("c")
        s = lax.axis_index("s")
        wid = s * NC + c
        pltpu.sync_copy(dst_hbm.at[wid], dst_v)
        pltpu.sync_copy(ones_hbm, ones_v)
        pltpu.sync_copy(zeros_hbm, zer_v)
        base = s * rows_per_tile
        for r in range(rows_per_tile // ZR):
            pltpu.sync_copy(zer_v, acc.at[pl.ds(base + r * ZR, ZR)])
        plsc.subcore_barrier()

        def issue(k, b):
            pltpu.async_copy(ones_v, acc.at[dst_v.at[k]], sems[b], add=True)

        def wait(k, b):
            pltpu.make_async_copy(ones_v, acc.at[dst_v.at[k]], sems[b]).wait()

        def group(g, first):
            for b in range(NSEM):
                k = g * NSEM + b
                if not first:
                    wait(k - NSEM, b)
                issue(k, b)

        group(0, True)
        if n_groups > 1:
            @pl.loop(1, n_groups)
            def _(g):
                group(g, False)
        for b in range(NSEM):
            wait(n_chunks - NSEM + b, b)

        plsc.subcore_barrier()
        pltpu.sync_copy(acc.at[pl.ds(base, rows_per_tile)],
                        out_hbm.at[c, pl.ds(base, rows_per_tile)])

    return body(dstp, ones_blk, zeros_blk)


def kernel(x, edge_index, W1, b1, W2, b2, W_mu, b_mu, W_lv, b_lv):
    N, _ = x.shape
    F1 = W1.shape[1]
    x = x.astype(jnp.float32)
    src = edge_index[0].astype(jnp.int32)
    dst = edge_index[1].astype(jnp.int32)
    E = src.shape[0]

    NP = -(-N // (NS * ZR)) * (NS * ZR)
    n_chunks = -(-E // (NW * CHUNK))
    n_chunks = -(-n_chunks // RING_LCM) * RING_LCM
    e_pad = NW * n_chunks * CHUNK - E
    pad_idx = N + (jnp.arange(e_pad, dtype=jnp.int32) % (NP - N))
    srcp = jnp.concatenate([src, pad_idx]).reshape(NW, n_chunks, CHUNK)
    dstp = jnp.concatenate([dst, pad_idx]).reshape(NW, n_chunks, CHUNK)

    ones16 = jnp.ones((CHUNK, 16), jnp.float32)
    z16 = jnp.zeros((ZR, 16), jnp.float32)
    zF1 = jnp.zeros((ZR, F1), jnp.float32)
    zF2 = jnp.zeros((ZR, 2 * F1), jnp.float32)

    Wml = jnp.concatenate([W_mu, W_lv], axis=1)
    bml = jnp.concatenate([b_mu, b_lv]).reshape(1, 2 * F1)
    b1r = b1.reshape(1, F1)
    b2r = b2.reshape(1, F1)

    degp = _sc_degree_pass(dstp, ones16, z16, NP, n_chunks)

    def tc1a(x_ref, w_ref, h_ref):
        h_ref[...] = jnp.dot(x_ref[...], w_ref[...],
                             preferred_element_type=jnp.float32)

    h1raw = pl.pallas_call(
        tc1a, out_shape=jax.ShapeDtypeStruct((N, F1), jnp.float32),
    )(x, W1)

    def tc1b(h_ref, degp_ref, t_ref, dinv_ref):
        deg = degp_ref[0, :, 0:1] + degp_ref[1, :, 0:1] + 1.0
        dinv = lax.rsqrt(deg)
        dinv_ref[...] = dinv
        t_ref[0:N, :] = h_ref[...] * dinv[0:N]
        t_ref[N:NP, :] = jnp.zeros((NP - N, F1), jnp.float32)

    table1, dinv = pl.pallas_call(
        tc1b,
        out_shape=(jax.ShapeDtypeStruct((NP, F1), jnp.float32),
                   jax.ShapeDtypeStruct((NP, 1), jnp.float32)),
    )(h1raw, degp)

    p1 = _sc_message_pass(table1, srcp, dstp, zF1, n_chunks)

    def tc_mid(t_ref, p_ref, dinv_ref, w_ref, b_ref, out_ref):
        dinv = dinv_ref[...]
        acc = p_ref[0] + p_ref[1] + t_ref[...]
        h = jnp.maximum(acc * dinv + b_ref[...], 0.0)
        out_ref[...] = jnp.dot(h, w_ref[...],
                               preferred_element_type=jnp.float32) * dinv

    table2 = pl.pallas_call(
        tc_mid, out_shape=jax.ShapeDtypeStruct((NP, F1), jnp.float32),
    )(table1, p1, dinv, W2, b1r)

    p2 = _sc_message_pass(table2, srcp, dstp, zF1, n_chunks)

    table3 = pl.pallas_call(
        tc_mid, out_shape=jax.ShapeDtypeStruct((NP, 2 * F1), jnp.float32),
    )(table2, p2, dinv, Wml, b2r)

    p3 = _sc_message_pass(table3, srcp, dstp, zF2, n_chunks)

    def tc4(t_ref, p_ref, dinv_ref, b_ref, mu_ref, lv_ref):
        o = (p_ref[0] + p_ref[1] + t_ref[...]) * dinv_ref[...] + b_ref[...]
        mu_ref[...] = o[0:N, 0:F1]
        lv_ref[...] = o[0:N, F1:2 * F1]

    mu, logvar = pl.pallas_call(
        tc4,
        out_shape=(jax.ShapeDtypeStruct((N, F1), jnp.float32),
                   jax.ShapeDtypeStruct((N, F1), jnp.float32)),
    )(table3, p3, dinv, bml)

    return (mu, logvar)


# speedup vs baseline: 1.0407x; 1.0407x over previous
import functools

import jax
import jax.numpy as jnp
from jax import lax
from jax.experimental import pallas as pl
from jax.experimental.pallas import tpu as pltpu
from jax.experimental.pallas import tpu_sc as plsc

NC = 2
NS = 16
NW = NC * NS
CHUNK = 128
ZR = 128


RING_LCM = 40


def _sc_message_pass(table, srcp, dstp, zeros_blk, n_chunks):
    NP, F = table.shape
    rows_per_tile = NP // NS
    RING, LOOK = (10, 7) if F <= 32 else (8, 6)
    n_groups = n_chunks // RING
    mesh = plsc.VectorSubcoreMesh(core_axis_name="c", subcore_axis_name="s")

    @functools.partial(
        pl.kernel,
        out_type=jax.ShapeDtypeStruct((NC, NP, F), jnp.float32),
        mesh=mesh,
        scratch_types=[
            pltpu.VMEM_SHARED((NP, F), jnp.float32),
            pltpu.VMEM((n_chunks, CHUNK), jnp.int32),
            pltpu.VMEM((n_chunks, CHUNK), jnp.int32),
        ] + [pltpu.VMEM((CHUNK, F), jnp.float32) for _ in range(RING)]
          + [pltpu.SemaphoreType.DMA for _ in range(2 * RING)],
        compiler_params=pltpu.CompilerParams(use_tc_tiling_on_sc=False),
    )
    def body(table_hbm, src_hbm, dst_hbm, zeros_hbm, out_hbm,
             acc, src_v, dst_v, *bufs_and_sems):
        bufs = bufs_and_sems[:RING]
        gsem = bufs_and_sems[RING:2 * RING]
        ssem = bufs_and_sems[2 * RING:]
        c = lax.axis_index("c")
        s = lax.axis_index("s")
        wid = s * NC + c
        pltpu.sync_copy(src_hbm.at[wid], src_v)
        pltpu.sync_copy(dst_hbm.at[wid], dst_v)
        pltpu.sync_copy(zeros_hbm, bufs[0])
        base = s * rows_per_tile
        for r in range(rows_per_tile // ZR):
            pltpu.sync_copy(bufs[0], acc.at[pl.ds(base + r * ZR, ZR)])
        plsc.subcore_barrier()

        def issue_gather(k, b):
            pltpu.async_copy(table_hbm.at[src_v.at[k]], bufs[b], gsem[b])

        def wait_gather(k, b):
            pltpu.make_async_copy(table_hbm.at[src_v.at[k]], bufs[b],
                                  gsem[b]).wait()

        def issue_scatter(k, b):
            pltpu.async_copy(bufs[b], acc.at[dst_v.at[k]], ssem[b], add=True)

        def wait_scatter(k, b):
            pltpu.make_async_copy(bufs[b], acc.at[dst_v.at[k]],
                                  ssem[b]).wait()

        def group(g, first, last):
            base_k = g * RING
            for b in range(RING):
                k = base_k + b
                wait_gather(k, b)
                issue_scatter(k, b)
                bb = (b + LOOK) % RING
                if (not last) or (b < RING - LOOK):
                    if (not first) or (b >= RING - LOOK):
                        wait_scatter(k + LOOK - RING, bb)
                    issue_gather(k + LOOK, bb)

        for j in range(LOOK):
            issue_gather(j, j)
        group(0, True, n_groups == 1)
        if n_groups > 2:
            @pl.loop(1, n_groups - 1)
            def _(g):
                group(g, False, False)
        if n_groups > 1:
            group(n_groups - 1, False, True)
        for j in range(RING):
            wait_scatter(n_chunks - RING + j, j)

        plsc.subcore_barrier()
        pltpu.sync_copy(acc.at[pl.ds(base, rows_per_tile)],
                        out_hbm.at[c, pl.ds(base, rows_per_tile)])

    return body(table, srcp, dstp, zeros_blk)


def _sc_degree_pass(dstp, ones_blk, zeros_blk, NP, n_chunks):
    rows_per_tile = NP // NS
    mesh = plsc.VectorSubcoreMesh(core_axis_name="c", subcore_axis_name="s")

    NSEM = 4
    n_groups = n_chunks // NSEM

    @functools.partial(
        pl.kernel,
        out_type=jax.ShapeDtypeStruct((NC, NP, 16), jnp.float32),
        mesh=mesh,
        scratch_types=[
            pltpu.VMEM_SHARED((NP, 16), jnp.float32),
            pltpu.VMEM((n_chunks, CHUNK), jnp.int32),
            pltpu.VMEM((CHUNK, 16), jnp.float32),
            pltpu.VMEM((ZR, 16), jnp.float32),
        ] + [pltpu.SemaphoreType.DMA for _ in range(NSEM)],
        compiler_params=pltpu.CompilerParams(use_tc_tiling_on_sc=False),
    )
    def body(dst_hbm, ones_hbm, zeros_hbm, out_hbm, acc, dst_v, ones_v, zer_v,
             *sems):
        c = lax.axis_index("c")
        s = lax.axis_index("s")
        wid = s * NC + c
        pltpu.sync_copy(dst_hbm.at[wid], dst_v)
        pltpu.sync_copy(ones_hbm, ones_v)
        pltpu.sync_copy(zeros_hbm, zer_v)
        base = s * rows_per_tile
        for r in range(rows_per_tile // ZR):
            pltpu.sync_copy(zer_v, acc.at[pl.ds(base + r * ZR, ZR)])
        plsc.subcore_barrier()

        def issue(k, b):
            pltpu.async_copy(ones_v, acc.at[dst_v.at[k]], sems[b], add=True)

        def wait(k, b):
            pltpu.make_async_copy(ones_v, acc.at[dst_v.at[k]], sems[b]).wait()

        def group(g, first):
            for b in range(NSEM):
                k = g * NSEM + b
                if not first:
                    wait(k - NSEM, b)
                issue(k, b)

        group(0, True)
        if n_groups > 1:
            @pl.loop(1, n_groups)
            def _(g):
                group(g, False)
        for b in range(NSEM):
            wait(n_chunks - NSEM + b, b)

        plsc.subcore_barrier()
        pltpu.sync_copy(acc.at[pl.ds(base, rows_per_tile)],
                        out_hbm.at[c, pl.ds(base, rows_per_tile)])

    return body(dstp, ones_blk, zeros_blk)


def kernel(x, edge_index, W1, b1, W2, b2, W_mu, b_mu, W_lv, b_lv):
    N, _ = x.shape
    F1 = W1.shape[1]
    x = x.astype(jnp.float32)
    src = edge_index[0].astype(jnp.int32)
    dst = edge_index[1].astype(jnp.int32)
    E = src.shape[0]

    NP = -(-N // (NS * ZR)) * (NS * ZR)
    n_chunks = -(-E // (NW * CHUNK))
    n_chunks = -(-n_chunks // RING_LCM) * RING_LCM
    e_pad = NW * n_chunks * CHUNK - E
    pad_idx = N + (jnp.arange(e_pad, dtype=jnp.int32) % (NP - N))
    srcp = jnp.concatenate([src, pad_idx]).reshape(NW, n_chunks, CHUNK)
    dstp = jnp.concatenate([dst, pad_idx]).reshape(NW, n_chunks, CHUNK)

    ones16 = jnp.ones((CHUNK, 16), jnp.float32)
    z16 = jnp.zeros((ZR, 16), jnp.float32)
    zF1 = jnp.zeros((ZR, F1), jnp.float32)
    zF2 = jnp.zeros((ZR, 2 * F1), jnp.float32)

    Wml = jnp.concatenate([W_mu, W_lv], axis=1)
    bml = jnp.concatenate([b_mu, b_lv]).reshape(1, 2 * F1)
    b1r = b1.reshape(1, F1)
    b2r = b2.reshape(1, F1)

    degp = _sc_degree_pass(dstp, ones16, z16, NP, n_chunks)

    def tc1a(x_ref, w_ref, h_ref):
        h_ref[...] = jnp.dot(x_ref[...], w_ref[...],
                             preferred_element_type=jnp.float32)

    h1raw = pl.pallas_call(
        tc1a, out_shape=jax.ShapeDtypeStruct((N, F1), jnp.float32),
    )(x, W1)

    def tc1b(h_ref, degp_ref, t_ref, dinv_ref):
        deg = degp_ref[0, :, 0:1] + degp_ref[1, :, 0:1] + 1.0
        dinv = lax.rsqrt(deg)
        dinv_ref[...] = dinv
        t_ref[0:N, :] = h_ref[...] * dinv[0:N]
        t_ref[N:NP, :] = jnp.zeros((NP - N, F1), jnp.float32)

    table1, dinv = pl.pallas_call(
        tc1b,
        out_shape=(jax.ShapeDtypeStruct((NP, F1), jnp.float32),
                   jax.ShapeDtypeStruct((NP, 1), jnp.float32)),
    )(h1raw, degp)

    p1 = _sc_message_pass(table1, srcp, dstp, zF1, n_chunks)

    def tc_mid(t_ref, p_ref, dinv_ref, w_ref, b_ref, out_ref):
        dinv = dinv_ref[...]
        acc = p_ref[0] + p_ref[1] + t_ref[...]
        h = jnp.maximum(acc * dinv + b_ref[...], 0.0)
        out_ref[...] = jnp.dot(h, w_ref[...],
                               preferred_element_type=jnp.float32) * dinv

    table2 = pl.pallas_call(
        tc_mid, out_shape=jax.ShapeDtypeStruct((NP, F1), jnp.float32),
    )(table1, p1, dinv, W2, b1r)

    p2 = _sc_message_pass(table2, srcp, dstp, zF1, n_chunks)

    table3 = pl.pallas_call(
        tc_mid, out_shape=jax.ShapeDtypeStruct((NP, 2 * F1), jnp.float32),
    )(table2, p2, dinv, Wml, b2r)

    p3 = _sc_message_pass(table3, srcp, dstp, zF2, n_chunks)

    def tc4(t_ref, p_ref, dinv_ref, b_ref, mu_ref, lv_ref):
        o = (p_ref[0] + p_ref[1] + t_ref[...]) * dinv_ref[...] + b_ref[...]
        mu_ref[...] = o[0:N, 0:F1]
        lv_ref[...] = o[0:N, F1:2 * F1]

    mu, logvar = pl.pallas_call(
        tc4,
        out_shape=(jax.ShapeDtypeStruct((N, F1), jnp.float32),
                   jax.ShapeDtypeStruct((N, F1), jnp.float32)),
    )(table3, p3, dinv, bml)

    return (mu, logvar)


# speedup vs baseline: 1.0437x; 1.0029x over previous
import functools

import jax
import jax.numpy as jnp
from jax import lax
from jax.experimental import pallas as pl
from jax.experimental.pallas import tpu as pltpu
from jax.experimental.pallas import tpu_sc as plsc

NC = 2
NS = 16
NW = NC * NS
CHUNK = 128
ZR = 128


RING_LCM = 40


def _sc_message_pass(table, srcp, dstp, zeros_blk, n_chunks):
    NP, F = table.shape
    rows_per_tile = NP // NS
    RING, LOOK = (10, 7) if F <= 32 else (8, 6)
    n_groups = n_chunks // RING
    mesh = plsc.VectorSubcoreMesh(core_axis_name="c", subcore_axis_name="s")

    @functools.partial(
        pl.kernel,
        out_type=jax.ShapeDtypeStruct((NC, NP, F), jnp.float32),
        mesh=mesh,
        scratch_types=[
            pltpu.VMEM_SHARED((NP, F), jnp.float32),
            pltpu.VMEM((n_chunks, CHUNK), jnp.int32),
            pltpu.VMEM((n_chunks, CHUNK), jnp.int32),
        ] + [pltpu.VMEM((CHUNK, F), jnp.float32) for _ in range(RING)]
          + [pltpu.SemaphoreType.DMA for _ in range(2 * RING)],
        compiler_params=pltpu.CompilerParams(use_tc_tiling_on_sc=False),
    )
    def body(table_hbm, src_hbm, dst_hbm, zeros_hbm, out_hbm,
             acc, src_v, dst_v, *bufs_and_sems):
        bufs = bufs_and_sems[:RING]
        gsem = bufs_and_sems[RING:2 * RING]
        ssem = bufs_and_sems[2 * RING:]
        c = lax.axis_index("c")
        s = lax.axis_index("s")
        wid = s * NC + c
        pltpu.sync_copy(src_hbm.at[wid], src_v)
        pltpu.sync_copy(dst_hbm.at[wid], dst_v)
        pltpu.sync_copy(zeros_hbm, bufs[0])
        base = s * rows_per_tile
        for r in range(rows_per_tile // ZR):
            pltpu.sync_copy(bufs[0], acc.at[pl.ds(base + r * ZR, ZR)])
        plsc.subcore_barrier()

        def issue_gather(k, b):
            pltpu.async_copy(table_hbm.at[src_v.at[k]], bufs[b], gsem[b])

        def wait_gather(k, b):
            pltpu.make_async_copy(table_hbm.at[src_v.at[k]], bufs[b],
                                  gsem[b]).wait()

        def issue_scatter(k, b):
            pltpu.async_copy(bufs[b], acc.at[dst_v.at[k]], ssem[b], add=True)

        def wait_scatter(k, b):
            pltpu.make_async_copy(bufs[b], acc.at[dst_v.at[k]],
                                  ssem[b]).wait()

        def group(g, first, last):
            base_k = g * RING
            for b in range(RING):
                k = base_k + b
                wait_gather(k, b)
                issue_scatter(k, b)
                bb = (b + LOOK) % RING
                if (not last) or (b < RING - LOOK):
                    if (not first) or (b >= RING - LOOK):
                        wait_scatter(k + LOOK - RING, bb)
                    issue_gather(k + LOOK, bb)

        for j in range(LOOK):
            issue_gather(j, j)
        group(0, True, n_groups == 1)
        if n_groups > 2:
            @pl.loop(1, n_groups - 1)
            def _(g):
                group(g, False, False)
        if n_groups > 1:
            group(n_groups - 1, False, True)
        for j in range(RING):
            wait_scatter(n_chunks - RING + j, j)

        plsc.subcore_barrier()
        pltpu.sync_copy(acc.at[pl.ds(base, rows_per_tile)],
                        out_hbm.at[c, pl.ds(base, rows_per_tile)])

    return body(table, srcp, dstp, zeros_blk)


def _sc_degree_pass(dstp, ones_blk, zeros_blk, NP, n_chunks):
    rows_per_tile = NP // NS
    mesh = plsc.VectorSubcoreMesh(core_axis_name="c", subcore_axis_name="s")

    NSEM = 4
    n_groups = n_chunks // NSEM

    @functools.partial(
        pl.kernel,
        out_type=jax.ShapeDtypeStruct((NC, NP, 16), jnp.float32),
        mesh=mesh,
        scratch_types=[
            pltpu.VMEM_SHARED((NP, 16), jnp.float32),
            pltpu.VMEM((n_chunks, CHUNK), jnp.int32),
            pltpu.VMEM((CHUNK, 16), jnp.float32),
            pltpu.VMEM((ZR, 16), jnp.float32),
        ] + [pltpu.SemaphoreType.DMA for _ in range(NSEM)],
        compiler_params=pltpu.CompilerParams(use_tc_tiling_on_sc=False),
    )
    def body(dst_hbm, ones_hbm, zeros_hbm, out_hbm, acc, dst_v, ones_v, zer_v,
             *sems):
        c = lax.axis_index("c")
        s = lax.axis_index("s")
        wid = s * NC + c
        pltpu.sync_copy(dst_hbm.at[wid], dst_v)
        pltpu.sync_copy(ones_hbm, ones_v)
        pltpu.sync_copy(zeros_hbm, zer_v)
        base = s * rows_per_tile
        for r in range(rows_per_tile // ZR):
            pltpu.sync_copy(zer_v, acc.at[pl.ds(base + r * ZR, ZR)])
        plsc.subcore_barrier()

        def issue(k, b):
            pltpu.async_copy(ones_v, acc.at[dst_v.at[k]], sems[b], add=True)

        def wait(k, b):
            pltpu.make_async_copy(ones_v, acc.at[dst_v.at[k]], sems[b]).wait()

        def group(g, first):
            for b in range(NSEM):
                k = g * NSEM + b
                if not first:
                    wait(k - NSEM, b)
                issue(k, b)

        group(0, True)
        if n_groups > 1:
            @pl.loop(1, n_groups)
            def _(g):
                group(g, False)
        for b in range(NSEM):
            wait(n_chunks - NSEM + b, b)

        plsc.subcore_barrier()
        pltpu.sync_copy(acc.at[pl.ds(base, rows_per_tile)],
                        out_hbm.at[c, pl.ds(base, rows_per_tile)])

    return body(dstp, ones_blk, zeros_blk)


def kernel(x, edge_index, W1, b1, W2, b2, W_mu, b_mu, W_lv, b_lv):
    N, _ = x.shape
    F1 = W1.shape[1]
    x = x.astype(jnp.float32)
    src = edge_index[0].astype(jnp.int32)
    dst = edge_index[1].astype(jnp.int32)
    E = src.shape[0]

    NP = -(-(N + 1) // (NS * ZR)) * (NS * ZR)
    n_chunks = -(-E // (NW * CHUNK))
    n_chunks = -(-n_chunks // RING_LCM) * RING_LCM
    e_pad = NW * n_chunks * CHUNK - E
    pad_idx = N + (jnp.arange(e_pad, dtype=jnp.int32) % (NP - N))
    srcp = jnp.concatenate([src, pad_idx]).reshape(NW, n_chunks, CHUNK)
    dstp = jnp.concatenate([dst, pad_idx]).reshape(NW, n_chunks, CHUNK)

    ones16 = jnp.ones((CHUNK, 16), jnp.float32)
    z16 = jnp.zeros((ZR, 16), jnp.float32)
    zF1 = jnp.zeros((ZR, F1), jnp.float32)
    zF2 = jnp.zeros((ZR, 2 * F1), jnp.float32)

    Wml = jnp.concatenate([W_mu, W_lv], axis=1)
    bml = jnp.concatenate([b_mu, b_lv]).reshape(1, 2 * F1)
    b1r = b1.reshape(1, F1)
    b2r = b2.reshape(1, F1)

    degp = _sc_degree_pass(dstp, ones16, z16, NP, n_chunks)

    def tc1a(x_ref, w_ref, h_ref):
        h_ref[...] = jnp.dot(x_ref[...], w_ref[...],
                             preferred_element_type=jnp.float32)

    h1raw = pl.pallas_call(
        tc1a, out_shape=jax.ShapeDtypeStruct((N, F1), jnp.float32),
    )(x, W1)

    def tc1b(h_ref, degp_ref, t_ref, dinv_ref):
        deg = degp_ref[0, :, 0:1] + degp_ref[1, :, 0:1] + 1.0
        dinv = lax.rsqrt(deg)
        dinv_ref[...] = dinv
        t_ref[0:N, :] = h_ref[...] * dinv[0:N]
        t_ref[N:NP, :] = jnp.zeros((NP - N, F1), jnp.float32)

    table1, dinv = pl.pallas_call(
        tc1b,
        out_shape=(jax.ShapeDtypeStruct((NP, F1), jnp.float32),
                   jax.ShapeDtypeStruct((NP, 1), jnp.float32)),
    )(h1raw, degp)

    p1 = _sc_message_pass(table1, srcp, dstp, zF1, n_chunks)

    def tc_mid(t_ref, p_ref, dinv_ref, w_ref, b_ref, out_ref):
        dinv = dinv_ref[...]
        acc = p_ref[0] + p_ref[1] + t_ref[...]
        h = jnp.maximum(acc * dinv + b_ref[...], 0.0)
        out_ref[...] = jnp.dot(h, w_ref[...],
                               preferred_element_type=jnp.float32) * dinv

    table2 = pl.pallas_call(
        tc_mid, out_shape=jax.ShapeDtypeStruct((NP, F1), jnp.float32),
    )(table1, p1, dinv, W2, b1r)

    p2 = _sc_message_pass(table2, srcp, dstp, zF1, n_chunks)

    table3 = pl.pallas_call(
        tc_mid, out_shape=jax.ShapeDtypeStruct((NP, 2 * F1), jnp.float32),
    )(table2, p2, dinv, Wml, b2r)

    p3 = _sc_message_pass(table3, srcp, dstp, zF2, n_chunks)

    def tc4(t_ref, p_ref, dinv_ref, b_ref, mu_ref, lv_ref):
        o = (p_ref[0] + p_ref[1] + t_ref[...]) * dinv_ref[...] + b_ref[...]
        mu_ref[...] = o[0:N, 0:F1]
        lv_ref[...] = o[0:N, F1:2 * F1]

    mu, logvar = pl.pallas_call(
        tc4,
        out_shape=(jax.ShapeDtypeStruct((N, F1), jnp.float32),
                   jax.ShapeDtypeStruct((N, F1), jnp.float32)),
    )(table3, p3, dinv, bml)

    return (mu, logvar)
